# baseline (speedup 1.0000x reference)
import functools

import jax
import jax.numpy as jnp
from jax import lax
from jax.experimental import pallas as pl
from jax.experimental.pallas import tpu as pltpu

N_IN = 1000
N_PAD = 1024
NMS_N = 512
NMS_REAL = 500
OUT_PAD = 128
OUT_REAL = 100
H = 112
D = H * H
DH = D // 2
KB = 896
KSTEPS = 14
SIGMA = 2.0

F32 = jnp.float32
I32 = jnp.int32


def _sort_body(s_row_ref, s_col_ref, l_row_ref, idx_ref, ss_ref, ls_ref):
    s_row = s_row_ref[...]
    s_col = s_col_ref[...]
    l_row = l_row_ref[...]

    jj = lax.broadcasted_iota(I32, (N_PAD, N_PAD), 0)
    ii = lax.broadcasted_iota(I32, (N_PAD, N_PAD), 1)
    gt = (s_col > s_row).astype(I32)
    tie = ((s_col == s_row) & (jj < ii)).astype(I32)
    rank_row = jnp.sum(gt + tie, axis=0, keepdims=True)

    rr = lax.broadcasted_iota(I32, (NMS_N, N_PAD), 0)
    onehot = (rr == jnp.broadcast_to(rank_row, (NMS_N, N_PAD))).astype(I32)
    col_i = lax.broadcasted_iota(I32, (NMS_N, N_PAD), 1)

    idx_ref[...] = jnp.sum(onehot * col_i, axis=1, keepdims=True)
    s_bcast = jnp.broadcast_to(s_row, (NMS_N, N_PAD))
    ss_ref[...] = jnp.sum(
        jnp.where(onehot == 1, s_bcast, jnp.zeros_like(s_bcast)),
        axis=1, keepdims=True)
    ls_ref[...] = jnp.sum(onehot * l_row, axis=1, keepdims=True)


def _run_sort(scores_p, labels_p):
    return pl.pallas_call(
        _sort_body,
        out_shape=[
            jax.ShapeDtypeStruct((NMS_N, 1), I32),
            jax.ShapeDtypeStruct((NMS_N, 1), F32),
            jax.ShapeDtypeStruct((NMS_N, 1), I32),
        ],
    )(scores_p.reshape(1, N_PAD), scores_p.reshape(N_PAD, 1),
      labels_p.reshape(1, N_PAD))


@functools.lru_cache(maxsize=None)
def _make_sc_gather(n_rows, rows_per_tile, chunk, n_buf, width=D):
    from jax.experimental.pallas import tpu_sc as plsc

    mesh = plsc.VectorSubcoreMesh(core_axis_name="c", subcore_axis_name="s")
    n_tiles = n_rows // rows_per_tile
    n_chunks = rows_per_tile // chunk

    @functools.partial(
        pl.kernel,
        mesh=mesh,
        out_type=jax.ShapeDtypeStruct((n_rows, width), F32),
        scratch_types=[
            pltpu.VMEM((n_chunks, 8), I32),
            *([pltpu.VMEM((chunk, width), F32)] * n_buf),
            *([pltpu.SemaphoreType.DMA] * n_buf),
            *([pltpu.SemaphoreType.DMA] * n_buf),
        ],
    )
    def gather_k(masks_hbm, idx_hbm, out_hbm, idx_v, *bufs_sems):
        bufs = bufs_sems[:n_buf]
        gsems = bufs_sems[n_buf:2 * n_buf]
        ssems = bufs_sems[2 * n_buf:]
        wid = lax.axis_index("s") * 2 + lax.axis_index("c")

        @pl.when(wid < n_tiles)
        def _():
            base = wid * rows_per_tile
            pltpu.sync_copy(idx_hbm.at[wid], idx_v)

            def gather_start(c, b):
                return pltpu.make_async_copy(
                    masks_hbm.at[idx_v.at[c, pl.ds(0, chunk)]],
                    bufs[b], gsems[b])

            def scatter_start(c, b):
                return pltpu.make_async_copy(
                    bufs[b], out_hbm.at[pl.ds(base + c * chunk, chunk)],
                    ssems[b])

            for c in range(min(n_buf, n_chunks)):
                gather_start(c, c).start()
            for c in range(n_chunks):
                b = c % n_buf
                gather_start(c, b).wait()
                scatter_start(c, b).start()
                nxt = c + n_buf
                if nxt < n_chunks:
                    scatter_start(c, b).wait()
                    gather_start(nxt, b).start()
            for c in range(max(0, n_chunks - n_buf), n_chunks):
                scatter_start(c, c % n_buf).wait()

    return gather_k


def _nms_body(ss_col_ref, ss_row_ref, ls_col_ref, ls_row_ref, si_row_ref,
              ylo_ref, yhi_ref, sf_ref, lf_ref, kf_ref,
              inter_ref, acol_ref, arow_ref):
    k = pl.program_id(0)

    @pl.when(k == 0)
    def _():
        inter_ref[...] = jnp.zeros_like(inter_ref)
        acol_ref[...] = jnp.zeros_like(acol_ref)
        arow_ref[...] = jnp.zeros_like(arow_ref)

    nt = (((1,), (1,)), ((), ()))
    ones_row = jnp.ones((1, KB), F32)

    def acc(y):
        inter_ref[...] += lax.dot_general(y, y, nt, preferred_element_type=F32)
        acol_ref[...] += lax.dot_general(y, ones_row, nt,
                                         preferred_element_type=F32)
        arow_ref[...] += lax.dot_general(ones_row, y, nt,
                                         preferred_element_type=F32)

    @pl.when(k < KSTEPS // 2)
    def _():
        acc(ylo_ref[...])

    @pl.when(k >= KSTEPS // 2)
    def _():
        acc(yhi_ref[...])

    @pl.when(k == KSTEPS - 1)
    def _():
        n = NMS_N
        inter = inter_ref[...]
        a_col = acol_ref[...]
        a_row = arow_ref[...]
        ls_col = ls_col_ref[...]
        ls_row = ls_row_ref[...]
        ss_col = ss_col_ref[...]
        ss_row = ss_row_ref[...]

        ii = lax.broadcasted_iota(I32, (n, n), 0)
        jj = lax.broadcasted_iota(I32, (n, n), 1)
        iou = inter / (a_col + a_row - inter)
        label_eq = ls_col == ls_row
        zero = jnp.zeros((n, n), F32)
        d = jnp.where((ii < jj) & label_eq & (jj < NMS_REAL), iou, zero)
        d_t = d.T

        c_row = jnp.max(d, axis=0, keepdims=True)
        c_col = jnp.max(d_t, axis=1, keepdims=True)

        neg = jnp.full((n, n), -jnp.inf, F32)
        e = jnp.where(ii < NMS_REAL, d * d - c_col * c_col, neg)
        e_t = jnp.where(jj < NMS_REAL, d_t * d_t - c_row * c_row, neg)
        max_e_row = jnp.max(e, axis=0, keepdims=True)
        max_e_col = jnp.max(e_t, axis=1, keepdims=True)

        iota_row = lax.broadcasted_iota(I32, (1, n), 1)
        iota_col = lax.broadcasted_iota(I32, (n, 1), 0)
        su_row = jnp.where(iota_row < NMS_REAL,
                           ss_row * jnp.exp(-SIGMA * max_e_row),
                           jnp.full((1, n), -1.0, F32))
        su_col = jnp.where(iota_col < NMS_REAL,
                           ss_col * jnp.exp(-SIGMA * max_e_col),
                           jnp.full((n, 1), -1.0, F32))

        gt2 = (su_col > su_row).astype(I32)
        tie2 = ((su_col == su_row) & (ii < jj)).astype(I32)
        rank2_row = jnp.sum(gt2 + tie2, axis=0, keepdims=True)

        rr = lax.broadcasted_iota(I32, (OUT_PAD, n), 0)
        onehot = (rr == jnp.broadcast_to(rank2_row, (OUT_PAD, n))).astype(I32)
        sf_ref[...] = jnp.sum(onehot.astype(F32) * su_row, axis=1,
                              keepdims=True)
        lf_ref[...] = jnp.sum(onehot * ls_row, axis=1, keepdims=True)
        kf_ref[...] = jnp.sum(onehot * si_row_ref[...], axis=1, keepdims=True)


def _run_nms(ss, ls, si, flat_lo, flat_hi):
    vec_spec = pl.BlockSpec((NMS_N, 1), lambda k: (0, 0))
    row_spec = pl.BlockSpec((1, NMS_N), lambda k: (0, 0))
    out_spec = pl.BlockSpec((OUT_PAD, 1), lambda k: (0, 0))
    half = KSTEPS // 2
    lo_spec = pl.BlockSpec((NMS_N, KB),
                           lambda k: (0, jnp.minimum(k, half - 1)))
    hi_spec = pl.BlockSpec((NMS_N, KB),
                           lambda k: (0, jnp.maximum(k - half, 0)))
    return pl.pallas_call(
        _nms_body,
        grid=(KSTEPS,),
        in_specs=[vec_spec, row_spec, vec_spec, row_spec, row_spec,
                  lo_spec, hi_spec],
        out_specs=[out_spec, out_spec, out_spec],
        out_shape=[
            jax.ShapeDtypeStruct((OUT_PAD, 1), F32),
            jax.ShapeDtypeStruct((OUT_PAD, 1), I32),
            jax.ShapeDtypeStruct((OUT_PAD, 1), I32),
        ],
        scratch_shapes=[
            pltpu.VMEM((NMS_N, NMS_N), F32),
            pltpu.VMEM((NMS_N, 1), F32),
            pltpu.VMEM((1, NMS_N), F32),
        ],
    )(ss.reshape(NMS_N, 1), ss.reshape(1, NMS_N),
      ls.reshape(NMS_N, 1), ls.reshape(1, NMS_N),
      si.reshape(1, NMS_N), flat_lo, flat_hi)


def _pad_idx(idx, rows_per_tile, chunk):
    n = idx.shape[0]
    rows = idx.reshape(n // chunk, chunk)
    rows = jnp.pad(rows, ((0, 0), (0, 8 - chunk)))
    return rows.reshape(n // rows_per_tile, rows_per_tile // chunk, 8)


@jax.jit
def kernel(masks, labels, scores):
    mflat_lo = masks[:, :H // 2, :].reshape(N_IN, DH)
    mflat_hi = masks[:, H // 2:, :].reshape(N_IN, DH)
    s_p = jnp.pad(scores, (0, N_PAD - N_IN), constant_values=-jnp.inf)
    l_p = jnp.pad(labels, (0, N_PAD - N_IN))

    si, ss, ls = _run_sort(s_p, l_p)
    si_p = _pad_idx(si.reshape(NMS_N), 16, 8)
    half_g = _make_sc_gather(NMS_N, 16, 8, 2, DH)
    flat_lo = half_g(mflat_lo, si_p)
    flat_hi = half_g(mflat_hi, si_p)
    sf, lf, kf = _run_nms(ss, ls, si, flat_lo, flat_hi)
    kf_flat = kf.reshape(OUT_PAD)
    kf_p = _pad_idx(kf_flat, 4, 4)
    out_g = _make_sc_gather(OUT_PAD, 4, 4, 1, DH)
    m_lo = out_g(mflat_lo, kf_p)
    m_hi = out_g(mflat_hi, kf_p)
    m_out = jnp.concatenate(
        [m_lo.reshape(OUT_PAD, H // 2, H), m_hi.reshape(OUT_PAD, H // 2, H)],
        axis=1)

    return (sf[:OUT_REAL, 0], lf[:OUT_REAL, 0],
            m_out[:OUT_REAL], kf_flat[:OUT_REAL])


# speedup vs baseline: 1.3951x; 1.3951x over previous
import functools

import jax
import jax.numpy as jnp
from jax import lax
from jax.experimental import pallas as pl
from jax.experimental.pallas import tpu as pltpu

N_IN = 1000
N_PAD = 1024
NMS_N = 512
NMS_REAL = 500
OUT_PAD = 128
OUT_REAL = 100
H = 112
D = H * H
KB = 1792
KSTEPS = 7
SIGMA = 2.0

F32 = jnp.float32
I32 = jnp.int32


def _sort_body(s_row_ref, s_col_ref, l_row_ref, idx_ref, ss_ref, ls_ref):
    s_row = s_row_ref[...]
    s_col = s_col_ref[...]
    l_row = l_row_ref[...]

    jj = lax.broadcasted_iota(I32, (N_PAD, N_PAD), 0)
    ii = lax.broadcasted_iota(I32, (N_PAD, N_PAD), 1)
    gt = (s_col > s_row).astype(I32)
    tie = ((s_col == s_row) & (jj < ii)).astype(I32)
    rank_row = jnp.sum(gt + tie, axis=0, keepdims=True)

    rr = lax.broadcasted_iota(I32, (NMS_N, N_PAD), 0)
    onehot = (rr == jnp.broadcast_to(rank_row, (NMS_N, N_PAD))).astype(I32)
    col_i = lax.broadcasted_iota(I32, (NMS_N, N_PAD), 1)

    idx_ref[...] = jnp.sum(onehot * col_i, axis=1, keepdims=True)
    s_bcast = jnp.broadcast_to(s_row, (NMS_N, N_PAD))
    ss_ref[...] = jnp.sum(
        jnp.where(onehot == 1, s_bcast, jnp.zeros_like(s_bcast)),
        axis=1, keepdims=True)
    ls_ref[...] = jnp.sum(onehot * l_row, axis=1, keepdims=True)


def _run_sort(scores_p, labels_p):
    return pl.pallas_call(
        _sort_body,
        out_shape=[
            jax.ShapeDtypeStruct((NMS_N, 1), I32),
            jax.ShapeDtypeStruct((NMS_N, 1), F32),
            jax.ShapeDtypeStruct((NMS_N, 1), I32),
        ],
    )(scores_p.reshape(1, N_PAD), scores_p.reshape(N_PAD, 1),
      labels_p.reshape(1, N_PAD))


@functools.lru_cache(maxsize=None)
def _make_sc_gather(n_rows, rows_per_tile, chunk, n_buf, width=D):
    from jax.experimental.pallas import tpu_sc as plsc

    mesh = plsc.VectorSubcoreMesh(core_axis_name="c", subcore_axis_name="s")
    n_tiles = n_rows // rows_per_tile
    n_chunks = rows_per_tile // chunk

    @functools.partial(
        pl.kernel,
        mesh=mesh,
        out_type=jax.ShapeDtypeStruct((n_rows, width), F32),
        scratch_types=[
            pltpu.VMEM((n_chunks, 8), I32),
            *([pltpu.VMEM((chunk, width), F32)] * n_buf),
            *([pltpu.SemaphoreType.DMA] * n_buf),
            *([pltpu.SemaphoreType.DMA] * n_buf),
        ],
    )
    def gather_k(masks_hbm, idx_hbm, out_hbm, idx_v, *bufs_sems):
        bufs = bufs_sems[:n_buf]
        gsems = bufs_sems[n_buf:2 * n_buf]
        ssems = bufs_sems[2 * n_buf:]
        wid = lax.axis_index("s") * 2 + lax.axis_index("c")

        @pl.when(wid < n_tiles)
        def _():
            base = wid * rows_per_tile
            pltpu.sync_copy(idx_hbm.at[wid], idx_v)

            def gather_start(c, b):
                return pltpu.make_async_copy(
                    masks_hbm.at[idx_v.at[c, pl.ds(0, chunk)]],
                    bufs[b], gsems[b])

            def scatter_start(c, b):
                return pltpu.make_async_copy(
                    bufs[b], out_hbm.at[pl.ds(base + c * chunk, chunk)],
                    ssems[b])

            for c in range(min(n_buf, n_chunks)):
                gather_start(c, c).start()
            for c in range(n_chunks):
                b = c % n_buf
                gather_start(c, b).wait()
                scatter_start(c, b).start()
                nxt = c + n_buf
                if nxt < n_chunks:
                    scatter_start(c, b).wait()
                    gather_start(nxt, b).start()
            for c in range(max(0, n_chunks - n_buf), n_chunks):
                scatter_start(c, c % n_buf).wait()

    return gather_k


def _nms_body(ss_col_ref, ss_row_ref, ls_col_ref, ls_row_ref, si_row_ref,
              y_ref, sf_ref, lf_ref, kf_ref, inter_ref, acol_ref, arow_ref):
    k = pl.program_id(0)

    @pl.when(k == 0)
    def _():
        inter_ref[...] = jnp.zeros_like(inter_ref)
        acol_ref[...] = jnp.zeros_like(acol_ref)
        arow_ref[...] = jnp.zeros_like(arow_ref)

    y = y_ref[...]
    nt = (((1,), (1,)), ((), ()))
    ones_row = jnp.ones((1, KB), F32)
    inter_ref[...] += lax.dot_general(y, y, nt, preferred_element_type=F32)
    acol_ref[...] += lax.dot_general(y, ones_row, nt,
                                     preferred_element_type=F32)
    arow_ref[...] += lax.dot_general(ones_row, y, nt,
                                     preferred_element_type=F32)

    @pl.when(k == KSTEPS - 1)
    def _():
        n = NMS_N
        inter = inter_ref[...]
        a_col = acol_ref[...]
        a_row = arow_ref[...]
        ls_col = ls_col_ref[...]
        ls_row = ls_row_ref[...]
        ss_col = ss_col_ref[...]
        ss_row = ss_row_ref[...]

        ii = lax.broadcasted_iota(I32, (n, n), 0)
        jj = lax.broadcasted_iota(I32, (n, n), 1)
        iou = inter / (a_col + a_row - inter)
        label_eq = ls_col == ls_row
        zero = jnp.zeros((n, n), F32)
        d = jnp.where((ii < jj) & label_eq & (jj < NMS_REAL), iou, zero)
        d_t = d.T

        c_row = jnp.max(d, axis=0, keepdims=True)
        c_col = jnp.max(d_t, axis=1, keepdims=True)

        neg = jnp.full((n, n), -jnp.inf, F32)
        e = jnp.where(ii < NMS_REAL, d * d - c_col * c_col, neg)
        e_t = jnp.where(jj < NMS_REAL, d_t * d_t - c_row * c_row, neg)
        max_e_row = jnp.max(e, axis=0, keepdims=True)
        max_e_col = jnp.max(e_t, axis=1, keepdims=True)

        iota_row = lax.broadcasted_iota(I32, (1, n), 1)
        iota_col = lax.broadcasted_iota(I32, (n, 1), 0)
        su_row = jnp.where(iota_row < NMS_REAL,
                           ss_row * jnp.exp(-SIGMA * max_e_row),
                           jnp.full((1, n), -1.0, F32))
        su_col = jnp.where(iota_col < NMS_REAL,
                           ss_col * jnp.exp(-SIGMA * max_e_col),
                           jnp.full((n, 1), -1.0, F32))

        gt2 = (su_col > su_row).astype(I32)
        tie2 = ((su_col == su_row) & (ii < jj)).astype(I32)
        rank2_row = jnp.sum(gt2 + tie2, axis=0, keepdims=True)

        rr = lax.broadcasted_iota(I32, (OUT_PAD, n), 0)
        onehot = (rr == jnp.broadcast_to(rank2_row, (OUT_PAD, n))).astype(I32)
        sf_ref[...] = jnp.sum(onehot.astype(F32) * su_row, axis=1,
                              keepdims=True)
        lf_ref[...] = jnp.sum(onehot * ls_row, axis=1, keepdims=True)
        kf_ref[...] = jnp.sum(onehot * si_row_ref[...], axis=1, keepdims=True)


def _run_nms(ss, ls, si, flat_s):
    vec_spec = pl.BlockSpec((NMS_N, 1), lambda k: (0, 0))
    row_spec = pl.BlockSpec((1, NMS_N), lambda k: (0, 0))
    out_spec = pl.BlockSpec((OUT_PAD, 1), lambda k: (0, 0))
    return pl.pallas_call(
        _nms_body,
        grid=(KSTEPS,),
        in_specs=[vec_spec, row_spec, vec_spec, row_spec, row_spec,
                  pl.BlockSpec((NMS_N, KB), lambda k: (0, k))],
        out_specs=[out_spec, out_spec, out_spec],
        out_shape=[
            jax.ShapeDtypeStruct((OUT_PAD, 1), F32),
            jax.ShapeDtypeStruct((OUT_PAD, 1), I32),
            jax.ShapeDtypeStruct((OUT_PAD, 1), I32),
        ],
        scratch_shapes=[
            pltpu.VMEM((NMS_N, NMS_N), F32),
            pltpu.VMEM((NMS_N, 1), F32),
            pltpu.VMEM((1, NMS_N), F32),
        ],
    )(ss.reshape(NMS_N, 1), ss.reshape(1, NMS_N),
      ls.reshape(NMS_N, 1), ls.reshape(1, NMS_N),
      si.reshape(1, NMS_N), flat_s)


def _pad_idx(idx, rows_per_tile, chunk):
    n = idx.shape[0]
    rows = idx.reshape(n // chunk, chunk)
    rows = jnp.pad(rows, ((0, 0), (0, 8 - chunk)))
    return rows.reshape(n // rows_per_tile, rows_per_tile // chunk, 8)


@jax.jit
def kernel(masks, labels, scores):
    mflat = masks.reshape(N_IN, D)
    s_p = jnp.pad(scores, (0, N_PAD - N_IN), constant_values=-jnp.inf)
    l_p = jnp.pad(labels, (0, N_PAD - N_IN))

    si, ss, ls = _run_sort(s_p, l_p)
    si_p = _pad_idx(si.reshape(NMS_N), 16, 4)
    flat_s = _make_sc_gather(NMS_N, 16, 4, 2)(mflat, si_p)
    sf, lf, kf = _run_nms(ss, ls, si, flat_s)
    kf_flat = kf.reshape(OUT_PAD)
    m_out = _make_sc_gather(OUT_PAD, 4, 4, 1)(mflat, _pad_idx(kf_flat, 4, 4))

    return (sf[:OUT_REAL, 0], lf[:OUT_REAL, 0],
            m_out[:OUT_REAL].reshape(OUT_REAL, H, H), kf_flat[:OUT_REAL])
